# initial kernel scaffold (unmeasured)
import jax
import jax.numpy as jnp
from jax import lax
from jax.experimental import pallas as pl
from jax.experimental.pallas import tpu as pltpu

N_DEV = 32
LOG2 = 5
M = 512
D = 512


def kernel(partial, resid, gamma):
    partial2d = partial.reshape(M, D)
    gamma2d = gamma.reshape(1, D)

    def body(
        x_ref,
        resid_ref,
        gamma_ref,
        out_ref,
        rs_b0,
        rs_b1,
        rs_b2,
        rs_b3,
        rs_b4,
        ag_b0,
        ag_b1,
        ag_b2,
        ag_b3,
        ag_b4,
        rs_send,
        rs_recv,
        ag_send,
        ag_recv,
    ):
        me = lax.axis_index("i")
        rs_bufs = [rs_b0, rs_b1, rs_b2, rs_b3, rs_b4]
        ag_bufs = [ag_b0, ag_b1, ag_b2, ag_b3, ag_b4]

        out_ref[...] = x_ref[...]

        seg_start = jnp.int32(0)
        for k in range(LOG2):
            h = (M >> k) // 2
            partner = me ^ (1 << k)
            bit = (me >> k) & 1
            send_start = jnp.where(bit == 0, seg_start + h, seg_start)
            keep_start = jnp.where(bit == 0, seg_start, seg_start + h)
            rdma = pltpu.make_async_remote_copy(
                src_ref=out_ref.at[pl.ds(send_start, h)],
                dst_ref=rs_bufs[k],
                send_sem=rs_send.at[k],
                recv_sem=rs_recv.at[k],
                device_id=(partner,),
                device_id_type=pl.DeviceIdType.MESH,
            )
            rdma.start()
            rdma.wait()
            out_ref[pl.ds(keep_start, h)] = (
                out_ref[pl.ds(keep_start, h)] + rs_bufs[k][...]
            )
            seg_start = keep_start

        own_start = seg_start
        for k in range(LOG2 - 1, -1, -1):
            bsz = (M // 2) >> k
            partner = me ^ (1 << k)
            partner_start = own_start ^ bsz
            rdma = pltpu.make_async_remote_copy(
                src_ref=out_ref.at[pl.ds(own_start, bsz)],
                dst_ref=ag_bufs[LOG2 - 1 - k],
                send_sem=ag_send.at[k],
                recv_sem=ag_recv.at[k],
                device_id=(partner,),
                device_id_type=pl.DeviceIdType.MESH,
            )
            rdma.start()
            rdma.wait()
            out_ref[pl.ds(partner_start, bsz)] = ag_bufs[LOG2 - 1 - k][...]
            own_start = jnp.minimum(own_start, partner_start)

        y = out_ref[...] + resid_ref[...]
        ms = jnp.mean(y * y, axis=-1, keepdims=True)
        out_ref[...] = y * lax.rsqrt(ms + 1e-6) * gamma_ref[...]

    return pl.pallas_call(
        body,
        out_shape=jax.ShapeDtypeStruct((M, D), jnp.float32),
        in_specs=[
            pl.BlockSpec(memory_space=pltpu.VMEM),
            pl.BlockSpec(memory_space=pltpu.VMEM),
            pl.BlockSpec(memory_space=pltpu.VMEM),
        ],
        out_specs=pl.BlockSpec(memory_space=pltpu.VMEM),
        scratch_shapes=[
            pltpu.VMEM((256, D), jnp.float32),
            pltpu.VMEM((128, D), jnp.float32),
            pltpu.VMEM((64, D), jnp.float32),
            pltpu.VMEM((32, D), jnp.float32),
            pltpu.VMEM((16, D), jnp.float32),
            pltpu.VMEM((16, D), jnp.float32),
            pltpu.VMEM((32, D), jnp.float32),
            pltpu.VMEM((64, D), jnp.float32),
            pltpu.VMEM((128, D), jnp.float32),
            pltpu.VMEM((256, D), jnp.float32),
            pltpu.SemaphoreType.DMA((LOG2,)),
            pltpu.SemaphoreType.DMA((LOG2,)),
            pltpu.SemaphoreType.DMA((LOG2,)),
            pltpu.SemaphoreType.DMA((LOG2,)),
        ],
        compiler_params=pltpu.CompilerParams(collective_id=0),
    )(partial2d, resid, gamma2d)


# baseline (device time: 57555 ns/iter reference)
import jax
import jax.numpy as jnp
from jax import lax
from jax.experimental import pallas as pl
from jax.experimental.pallas import tpu as pltpu

N_DEV = 32
LOG2 = 5
M = 512
D = 512


def kernel(partial, resid, gamma):
    partial2d = partial.reshape(M, D)
    gamma2d = gamma.reshape(1, D)

    def body(
        x_ref,
        resid_ref,
        gamma_ref,
        out_ref,
        rs_b0,
        rs_b1,
        rs_b2,
        rs_b3,
        rs_b4,
        ag_b0,
        ag_b1,
        ag_b2,
        ag_b3,
        ag_b4,
        rs_send,
        rs_recv,
        ag_send,
        ag_recv,
    ):
        me = lax.axis_index("i")
        rs_bufs = [rs_b0, rs_b1, rs_b2, rs_b3, rs_b4]
        ag_bufs = [ag_b0, ag_b1, ag_b2, ag_b3, ag_b4]

        out_ref[...] = x_ref[...]

        seg_start = jnp.int32(0)
        for k in range(LOG2):
            h = (M >> k) // 2
            partner = me ^ (1 << k)
            bit = (me >> k) & 1
            send_start = pl.multiple_of(
                jnp.where(bit == 0, seg_start + h, seg_start), 8
            )
            keep_start = pl.multiple_of(
                jnp.where(bit == 0, seg_start, seg_start + h), 8
            )
            rdma = pltpu.make_async_remote_copy(
                src_ref=out_ref.at[pl.ds(send_start, h)],
                dst_ref=rs_bufs[k],
                send_sem=rs_send.at[k],
                recv_sem=rs_recv.at[k],
                device_id=(partner,),
                device_id_type=pl.DeviceIdType.MESH,
            )
            rdma.start()
            rdma.wait()
            out_ref[pl.ds(keep_start, h)] = (
                out_ref[pl.ds(keep_start, h)] + rs_bufs[k][...]
            )
            seg_start = keep_start

        own_start = seg_start
        for k in range(LOG2 - 1, -1, -1):
            bsz = (M // 2) >> k
            partner = me ^ (1 << k)
            own_start = pl.multiple_of(own_start, 8)
            partner_start = pl.multiple_of(own_start ^ bsz, 8)
            rdma = pltpu.make_async_remote_copy(
                src_ref=out_ref.at[pl.ds(own_start, bsz)],
                dst_ref=ag_bufs[LOG2 - 1 - k],
                send_sem=ag_send.at[k],
                recv_sem=ag_recv.at[k],
                device_id=(partner,),
                device_id_type=pl.DeviceIdType.MESH,
            )
            rdma.start()
            rdma.wait()
            out_ref[pl.ds(partner_start, bsz)] = ag_bufs[LOG2 - 1 - k][...]
            own_start = jnp.minimum(own_start, partner_start)

        y = out_ref[...] + resid_ref[...]
        ms = jnp.mean(y * y, axis=-1, keepdims=True)
        out_ref[...] = y * lax.rsqrt(ms + 1e-6) * gamma_ref[...]

    return pl.pallas_call(
        body,
        out_shape=jax.ShapeDtypeStruct((M, D), jnp.float32),
        in_specs=[
            pl.BlockSpec(memory_space=pltpu.VMEM),
            pl.BlockSpec(memory_space=pltpu.VMEM),
            pl.BlockSpec(memory_space=pltpu.VMEM),
        ],
        out_specs=pl.BlockSpec(memory_space=pltpu.VMEM),
        scratch_shapes=[
            pltpu.VMEM((256, D), jnp.float32),
            pltpu.VMEM((128, D), jnp.float32),
            pltpu.VMEM((64, D), jnp.float32),
            pltpu.VMEM((32, D), jnp.float32),
            pltpu.VMEM((16, D), jnp.float32),
            pltpu.VMEM((16, D), jnp.float32),
            pltpu.VMEM((32, D), jnp.float32),
            pltpu.VMEM((64, D), jnp.float32),
            pltpu.VMEM((128, D), jnp.float32),
            pltpu.VMEM((256, D), jnp.float32),
            pltpu.SemaphoreType.DMA((LOG2,)),
            pltpu.SemaphoreType.DMA((LOG2,)),
            pltpu.SemaphoreType.DMA((LOG2,)),
            pltpu.SemaphoreType.DMA((LOG2,)),
        ],
    )(partial2d, resid, gamma2d)


# device time: 51034 ns/iter; 1.1278x vs baseline; 1.1278x over previous
import jax
import jax.numpy as jnp
from jax import lax
from jax.experimental import pallas as pl
from jax.experimental.pallas import tpu as pltpu

N_DEV = 32
LOG2 = 5
M = 512
D = 512


def kernel(partial, resid, gamma):
    partial2d = partial.reshape(M, D)
    gamma2d = gamma.reshape(1, D)

    def body(
        x_ref,
        resid_ref,
        gamma_ref,
        out_ref,
        rs_b0,
        rs_b1,
        rs_b2,
        rs_b3,
        rs_b4,
        ag_b0,
        ag_b1,
        ag_b2,
        ag_b3,
        ag_b4,
        rs_send,
        rs_recv,
        ag_send,
        ag_recv,
    ):
        me = lax.axis_index("i")
        rs_bufs = [rs_b0, rs_b1, rs_b2, rs_b3, rs_b4]
        ag_bufs = [ag_b0, ag_b1, ag_b2, ag_b3, ag_b4]

        out_ref[...] = x_ref[...]

        barrier_sem = pltpu.get_barrier_semaphore()
        for k in range(LOG2):
            pl.semaphore_signal(
                barrier_sem,
                inc=1,
                device_id=(me ^ (1 << k),),
                device_id_type=pl.DeviceIdType.MESH,
            )
        pl.semaphore_wait(barrier_sem, LOG2)

        seg_start = jnp.int32(0)
        rs_rdmas = []
        for k in range(LOG2):
            h = (M >> k) // 2
            partner = me ^ (1 << k)
            bit = (me >> k) & 1
            send_start = pl.multiple_of(
                jnp.where(bit == 0, seg_start + h, seg_start), 8
            )
            keep_start = pl.multiple_of(
                jnp.where(bit == 0, seg_start, seg_start + h), 8
            )
            rdma = pltpu.make_async_remote_copy(
                src_ref=out_ref.at[pl.ds(send_start, h)],
                dst_ref=rs_bufs[k],
                send_sem=rs_send.at[k],
                recv_sem=rs_recv.at[k],
                device_id=(partner,),
                device_id_type=pl.DeviceIdType.MESH,
            )
            rdma.start()
            rdma.wait_recv()
            out_ref[pl.ds(keep_start, h)] = (
                out_ref[pl.ds(keep_start, h)] + rs_bufs[k][...]
            )
            rs_rdmas.append(rdma)
            seg_start = keep_start

        for rdma in rs_rdmas:
            rdma.wait_send()

        own = pl.multiple_of(seg_start, 8)
        rows = M // N_DEV
        y = out_ref[pl.ds(own, rows)] + resid_ref[pl.ds(own, rows)]
        ms = jnp.mean(y * y, axis=-1, keepdims=True)
        out_ref[pl.ds(own, rows)] = y * lax.rsqrt(ms + 1e-6) * gamma_ref[...]

        own_start = seg_start
        ag_rdmas = []
        for k in range(LOG2 - 1, -1, -1):
            bsz = (M // 2) >> k
            partner = me ^ (1 << k)
            own_start = pl.multiple_of(own_start, 8)
            partner_start = pl.multiple_of(own_start ^ bsz, 8)
            rdma = pltpu.make_async_remote_copy(
                src_ref=out_ref.at[pl.ds(own_start, bsz)],
                dst_ref=ag_bufs[LOG2 - 1 - k],
                send_sem=ag_send.at[k],
                recv_sem=ag_recv.at[k],
                device_id=(partner,),
                device_id_type=pl.DeviceIdType.MESH,
            )
            rdma.start()
            rdma.wait_recv()
            out_ref[pl.ds(partner_start, bsz)] = ag_bufs[LOG2 - 1 - k][...]
            ag_rdmas.append(rdma)
            own_start = jnp.minimum(own_start, partner_start)

        for rdma in ag_rdmas:
            rdma.wait_send()

    return pl.pallas_call(
        body,
        out_shape=jax.ShapeDtypeStruct((M, D), jnp.float32),
        in_specs=[
            pl.BlockSpec(memory_space=pltpu.VMEM),
            pl.BlockSpec(memory_space=pltpu.VMEM),
            pl.BlockSpec(memory_space=pltpu.VMEM),
        ],
        out_specs=pl.BlockSpec(memory_space=pltpu.VMEM),
        scratch_shapes=[
            pltpu.VMEM((256, D), jnp.float32),
            pltpu.VMEM((128, D), jnp.float32),
            pltpu.VMEM((64, D), jnp.float32),
            pltpu.VMEM((32, D), jnp.float32),
            pltpu.VMEM((16, D), jnp.float32),
            pltpu.VMEM((16, D), jnp.float32),
            pltpu.VMEM((32, D), jnp.float32),
            pltpu.VMEM((64, D), jnp.float32),
            pltpu.VMEM((128, D), jnp.float32),
            pltpu.VMEM((256, D), jnp.float32),
            pltpu.SemaphoreType.DMA((LOG2,)),
            pltpu.SemaphoreType.DMA((LOG2,)),
            pltpu.SemaphoreType.DMA((LOG2,)),
            pltpu.SemaphoreType.DMA((LOG2,)),
        ],
        compiler_params=pltpu.CompilerParams(collective_id=0),
    )(partial2d, resid, gamma2d)


# device time: 45931 ns/iter; 1.2531x vs baseline; 1.1111x over previous
import jax
import jax.numpy as jnp
from jax import lax
from jax.experimental import pallas as pl
from jax.experimental.pallas import tpu as pltpu

N_DEV = 32
LOG2 = 5
M = 512
D = 512

_OFF = {1: 0, 2: 1, 3: 3, 4: 6}
N_AG_SEMS = 15


def kernel(partial, resid, gamma):
    partial2d = partial.reshape(M, D)
    gamma2d = gamma.reshape(1, D)

    def body(
        x_ref,
        resid_ref,
        gamma_ref,
        out_ref,
        rs_b0,
        rs_b1,
        rs_b2,
        rs_b3,
        rs_b4,
        rs_send,
        rs_recv,
        ag_send,
        ag_recv,
    ):
        me = lax.axis_index("i")
        rs_bufs = [rs_b0, rs_b1, rs_b2, rs_b3, rs_b4]

        out_ref[...] = x_ref[...]

        barrier_sem = pltpu.get_barrier_semaphore()
        for k in range(LOG2):
            pl.semaphore_signal(
                barrier_sem,
                inc=1,
                device_id=(me ^ (1 << k),),
                device_id_type=pl.DeviceIdType.MESH,
            )
        pl.semaphore_wait(barrier_sem, LOG2)

        seg_start = jnp.int32(0)
        rs_rdmas = []
        for k in range(LOG2):
            h = (M >> k) // 2
            partner = me ^ (1 << k)
            bit = (me >> k) & 1
            send_start = pl.multiple_of(
                jnp.where(bit == 0, seg_start + h, seg_start), 8
            )
            keep_start = pl.multiple_of(
                jnp.where(bit == 0, seg_start, seg_start + h), 8
            )
            rdma = pltpu.make_async_remote_copy(
                src_ref=out_ref.at[pl.ds(send_start, h)],
                dst_ref=rs_bufs[k],
                send_sem=rs_send.at[k],
                recv_sem=rs_recv.at[k],
                device_id=(partner,),
                device_id_type=pl.DeviceIdType.MESH,
            )
            rdma.start()
            rdma.wait_recv()
            out_ref[pl.ds(keep_start, h)] = (
                out_ref[pl.ds(keep_start, h)] + rs_bufs[k][...]
            )
            rs_rdmas.append(rdma)
            seg_start = keep_start

        for rdma in rs_rdmas:
            rdma.wait_send()

        own16 = pl.multiple_of(seg_start, 8)
        rows = M // N_DEV
        y = out_ref[pl.ds(own16, rows)] + resid_ref[pl.ds(own16, rows)]
        ms = jnp.mean(y * y, axis=-1, keepdims=True)
        out_ref[pl.ds(own16, rows)] = y * lax.rsqrt(ms + 1e-6) * gamma_ref[...]

        a_desc = [None] * LOG2
        for k in range(LOG2 - 1, -1, -1):
            a_desc[k] = pltpu.make_async_remote_copy(
                src_ref=out_ref.at[pl.ds(own16, rows)],
                dst_ref=out_ref.at[pl.ds(own16, rows)],
                send_sem=ag_send.at[k],
                recv_sem=ag_recv.at[k],
                device_id=(me ^ (1 << k),),
                device_id_type=pl.DeviceIdType.MESH,
            )
            a_desc[k].start()

        f_desc = {}
        own_start = own16
        for k in range(LOG2 - 1, -1, -1):
            bsz = (M // 2) >> k
            own_start = pl.multiple_of(own_start, 8)
            r_start = pl.multiple_of(own_start ^ bsz, 8)
            a_desc[k].wait_recv()
            for j in range(LOG2 - 1, k, -1):
                f_desc[(k, j)].wait_recv()
            for kp in range(k - 1, -1, -1):
                idx = LOG2 + _OFF[k] + kp
                f = pltpu.make_async_remote_copy(
                    src_ref=out_ref.at[pl.ds(r_start, bsz)],
                    dst_ref=out_ref.at[pl.ds(r_start, bsz)],
                    send_sem=ag_send.at[idx],
                    recv_sem=ag_recv.at[idx],
                    device_id=(me ^ (1 << kp),),
                    device_id_type=pl.DeviceIdType.MESH,
                )
                f.start()
                f_desc[(kp, k)] = f
            own_start = jnp.minimum(own_start, r_start)

        for k in range(LOG2):
            a_desc[k].wait_send()
        for f in f_desc.values():
            f.wait_send()

    return pl.pallas_call(
        body,
        out_shape=jax.ShapeDtypeStruct((M, D), jnp.float32),
        in_specs=[
            pl.BlockSpec(memory_space=pltpu.VMEM),
            pl.BlockSpec(memory_space=pltpu.VMEM),
            pl.BlockSpec(memory_space=pltpu.VMEM),
        ],
        out_specs=pl.BlockSpec(memory_space=pltpu.VMEM),
        scratch_shapes=[
            pltpu.VMEM((256, D), jnp.float32),
            pltpu.VMEM((128, D), jnp.float32),
            pltpu.VMEM((64, D), jnp.float32),
            pltpu.VMEM((32, D), jnp.float32),
            pltpu.VMEM((16, D), jnp.float32),
            pltpu.SemaphoreType.DMA((LOG2,)),
            pltpu.SemaphoreType.DMA((LOG2,)),
            pltpu.SemaphoreType.DMA((N_AG_SEMS,)),
            pltpu.SemaphoreType.DMA((N_AG_SEMS,)),
        ],
        compiler_params=pltpu.CompilerParams(collective_id=0),
    )(partial2d, resid, gamma2d)


# device time: 43551 ns/iter; 1.3216x vs baseline; 1.0546x over previous
import jax
import jax.numpy as jnp
from jax import lax
from jax.experimental import pallas as pl
from jax.experimental.pallas import tpu as pltpu

N_DEV = 32
LOG2 = 5
M = 512
D = 512

_OFF = {1: 0, 2: 1, 3: 3, 4: 6}
N_AG_SEMS = 15


def kernel(partial, resid, gamma):
    partial2d = partial.reshape(M, D)
    gamma2d = gamma.reshape(1, D)

    def body(
        x_ref,
        resid_ref,
        gamma_ref,
        out_ref,
        rs_b0,
        rs_b1,
        rs_b2,
        rs_b3,
        rs_b4,
        rs_send,
        rs_recv,
        ag_send,
        ag_recv,
    ):
        me = lax.axis_index("i")
        rs_bufs = [rs_b0, rs_b1, rs_b2, rs_b3, rs_b4]

        out_ref[...] = x_ref[...]

        barrier_sem = pltpu.get_barrier_semaphore()
        for k in range(LOG2):
            pl.semaphore_signal(
                barrier_sem,
                inc=1,
                device_id=(me ^ (1 << k),),
                device_id_type=pl.DeviceIdType.MESH,
            )
        pl.semaphore_wait(barrier_sem, LOG2)

        def issue_rs(k, seg_start):
            h = (M >> k) // 2
            partner = me ^ (1 << k)
            bit = (me >> k) & 1
            send_start = pl.multiple_of(
                jnp.where(bit == 0, seg_start + h, seg_start), 8
            )
            keep_start = pl.multiple_of(
                jnp.where(bit == 0, seg_start, seg_start + h), 8
            )
            if k == LOG2 - 1:
                rdma = pltpu.make_async_remote_copy(
                    src_ref=out_ref.at[pl.ds(send_start, h)],
                    dst_ref=rs_bufs[k],
                    send_sem=rs_send.at[2 * k],
                    recv_sem=rs_recv.at[2 * k],
                    device_id=(partner,),
                    device_id_type=pl.DeviceIdType.MESH,
                )
                rdma.start()
                return [(rdma, jnp.int32(0), h)], keep_start
            hb = h // 2
            bitn = (me >> (k + 1)) & 1
            off1 = jnp.where(bitn == 0, hb, 0)
            off2 = hb - off1
            subs = []
            for c, off in ((0, off1), (1, off2)):
                off = pl.multiple_of(off, 8)
                rdma = pltpu.make_async_remote_copy(
                    src_ref=out_ref.at[pl.ds(send_start + off, hb)],
                    dst_ref=rs_bufs[k].at[pl.ds(c * hb, hb)],
                    send_sem=rs_send.at[2 * k + c],
                    recv_sem=rs_recv.at[2 * k + c],
                    device_id=(partner,),
                    device_id_type=pl.DeviceIdType.MESH,
                )
                rdma.start()
                subs.append((rdma, off, hb))
            return subs, keep_start

        rs_rdmas = []
        chunks, keep = issue_rs(0, jnp.int32(0))
        for k in range(LOG2):
            rdma1, o1, hb1 = chunks[0]
            rdma1.wait_recv()
            a1 = pl.multiple_of(keep + o1, 8)
            out_ref[pl.ds(a1, hb1)] = (
                out_ref[pl.ds(a1, hb1)] + rs_bufs[k][0:hb1, :]
            )
            rs_rdmas.append(rdma1)
            if k + 1 < LOG2:
                next_chunks, next_keep = issue_rs(k + 1, keep)
            if len(chunks) > 1:
                rdma2, o2, hb2 = chunks[1]
                rdma2.wait_recv()
                a2 = pl.multiple_of(keep + o2, 8)
                out_ref[pl.ds(a2, hb2)] = (
                    out_ref[pl.ds(a2, hb2)] + rs_bufs[k][hb2 : 2 * hb2, :]
                )
                rs_rdmas.append(rdma2)
            if k + 1 < LOG2:
                chunks, keep = next_chunks, next_keep
        seg_start = keep

        for rdma in rs_rdmas:
            rdma.wait_send()

        own16 = pl.multiple_of(seg_start, 8)
        rows = M // N_DEV
        y = out_ref[pl.ds(own16, rows)] + resid_ref[pl.ds(own16, rows)]
        ms = jnp.mean(y * y, axis=-1, keepdims=True)
        out_ref[pl.ds(own16, rows)] = y * lax.rsqrt(ms + 1e-6) * gamma_ref[...]

        a_desc = [None] * LOG2
        for k in range(LOG2 - 1, -1, -1):
            a_desc[k] = pltpu.make_async_remote_copy(
                src_ref=out_ref.at[pl.ds(own16, rows)],
                dst_ref=out_ref.at[pl.ds(own16, rows)],
                send_sem=ag_send.at[k],
                recv_sem=ag_recv.at[k],
                device_id=(me ^ (1 << k),),
                device_id_type=pl.DeviceIdType.MESH,
            )
            a_desc[k].start()

        f_desc = {}
        own_start = own16
        for k in range(LOG2 - 1, -1, -1):
            bsz = (M // 2) >> k
            own_start = pl.multiple_of(own_start, 8)
            r_start = pl.multiple_of(own_start ^ bsz, 8)
            a_desc[k].wait_recv()
            for j in range(LOG2 - 1, k, -1):
                f_desc[(k, j)].wait_recv()
            for kp in range(k - 1, -1, -1):
                idx = LOG2 + _OFF[k] + kp
                f = pltpu.make_async_remote_copy(
                    src_ref=out_ref.at[pl.ds(r_start, bsz)],
                    dst_ref=out_ref.at[pl.ds(r_start, bsz)],
                    send_sem=ag_send.at[idx],
                    recv_sem=ag_recv.at[idx],
                    device_id=(me ^ (1 << kp),),
                    device_id_type=pl.DeviceIdType.MESH,
                )
                f.start()
                f_desc[(kp, k)] = f
            own_start = jnp.minimum(own_start, r_start)

        for k in range(LOG2):
            a_desc[k].wait_send()
        for f in f_desc.values():
            f.wait_send()

    return pl.pallas_call(
        body,
        out_shape=jax.ShapeDtypeStruct((M, D), jnp.float32),
        in_specs=[
            pl.BlockSpec(memory_space=pltpu.VMEM),
            pl.BlockSpec(memory_space=pltpu.VMEM),
            pl.BlockSpec(memory_space=pltpu.VMEM),
        ],
        out_specs=pl.BlockSpec(memory_space=pltpu.VMEM),
        scratch_shapes=[
            pltpu.VMEM((256, D), jnp.float32),
            pltpu.VMEM((128, D), jnp.float32),
            pltpu.VMEM((64, D), jnp.float32),
            pltpu.VMEM((32, D), jnp.float32),
            pltpu.VMEM((16, D), jnp.float32),
            pltpu.SemaphoreType.DMA((2 * LOG2 - 1,)),
            pltpu.SemaphoreType.DMA((2 * LOG2 - 1,)),
            pltpu.SemaphoreType.DMA((N_AG_SEMS,)),
            pltpu.SemaphoreType.DMA((N_AG_SEMS,)),
        ],
        compiler_params=pltpu.CompilerParams(collective_id=0),
    )(partial2d, resid, gamma2d)


# device time: 40814 ns/iter; 1.4102x vs baseline; 1.0671x over previous
from itertools import combinations

import jax
import jax.numpy as jnp
from jax import lax
from jax.experimental import pallas as pl
from jax.experimental.pallas import tpu as pltpu

N_DEV = 32
LOG2 = 5
M = 512
D = 512

N_AG_SEMS = 31


def kernel(partial, resid, gamma):
    partial2d = partial.reshape(M, D)
    gamma2d = gamma.reshape(1, D)

    def body(
        x_ref,
        resid_ref,
        gamma_ref,
        out_ref,
        rs_b0,
        rs_b1,
        rs_b2,
        rs_b3,
        rs_b4,
        rs_send,
        rs_recv,
        ag_send,
        ag_recv,
    ):
        me = lax.axis_index("i")
        rs_bufs = [rs_b0, rs_b1, rs_b2, rs_b3, rs_b4]

        out_ref[...] = x_ref[...]

        barrier_sem = pltpu.get_barrier_semaphore()
        for k in range(LOG2):
            pl.semaphore_signal(
                barrier_sem,
                inc=1,
                device_id=(me ^ (1 << k),),
                device_id_type=pl.DeviceIdType.MESH,
            )
        pl.semaphore_wait(barrier_sem, LOG2)

        def issue_rs(k, seg_start):
            h = (M >> k) // 2
            partner = me ^ (1 << k)
            bit = (me >> k) & 1
            send_start = pl.multiple_of(
                jnp.where(bit == 0, seg_start + h, seg_start), 8
            )
            keep_start = pl.multiple_of(
                jnp.where(bit == 0, seg_start, seg_start + h), 8
            )
            if k == LOG2 - 1:
                rdma = pltpu.make_async_remote_copy(
                    src_ref=out_ref.at[pl.ds(send_start, h)],
                    dst_ref=rs_bufs[k],
                    send_sem=rs_send.at[2 * k],
                    recv_sem=rs_recv.at[2 * k],
                    device_id=(partner,),
                    device_id_type=pl.DeviceIdType.MESH,
                )
                rdma.start()
                return [(rdma, jnp.int32(0), h)], keep_start
            hb = h // 2
            bitn = (me >> (k + 1)) & 1
            off1 = jnp.where(bitn == 0, hb, 0)
            off2 = hb - off1
            subs = []
            for c, off in ((0, off1), (1, off2)):
                off = pl.multiple_of(off, 8)
                rdma = pltpu.make_async_remote_copy(
                    src_ref=out_ref.at[pl.ds(send_start + off, hb)],
                    dst_ref=rs_bufs[k].at[pl.ds(c * hb, hb)],
                    send_sem=rs_send.at[2 * k + c],
                    recv_sem=rs_recv.at[2 * k + c],
                    device_id=(partner,),
                    device_id_type=pl.DeviceIdType.MESH,
                )
                rdma.start()
                subs.append((rdma, off, hb))
            return subs, keep_start

        rs_rdmas = []
        chunks, keep = issue_rs(0, jnp.int32(0))
        for k in range(LOG2):
            rdma1, o1, hb1 = chunks[0]
            rdma1.wait_recv()
            a1 = pl.multiple_of(keep + o1, 8)
            out_ref[pl.ds(a1, hb1)] = (
                out_ref[pl.ds(a1, hb1)] + rs_bufs[k][0:hb1, :]
            )
            rs_rdmas.append(rdma1)
            if k + 1 < LOG2:
                next_chunks, next_keep = issue_rs(k + 1, keep)
            if len(chunks) > 1:
                rdma2, o2, hb2 = chunks[1]
                rdma2.wait_recv()
                a2 = pl.multiple_of(keep + o2, 8)
                out_ref[pl.ds(a2, hb2)] = (
                    out_ref[pl.ds(a2, hb2)] + rs_bufs[k][hb2 : 2 * hb2, :]
                )
                rs_rdmas.append(rdma2)
            if k + 1 < LOG2:
                chunks, keep = next_chunks, next_keep
        seg_start = keep

        for rdma in rs_rdmas:
            rdma.wait_send()

        own16 = pl.multiple_of(seg_start, 8)
        rows = M // N_DEV
        y = out_ref[pl.ds(own16, rows)] + resid_ref[pl.ds(own16, rows)]
        ms = jnp.mean(y * y, axis=-1, keepdims=True)
        out_ref[pl.ds(own16, rows)] = y * lax.rsqrt(ms + 1e-6) * gamma_ref[...]

        piece_ids = []
        for k in range(LOG2):
            for r in range(LOG2 - k):
                for S in combinations(range(k + 1, LOG2), r):
                    piece_ids.append((k, S))
        sem_idx = {pid: n for n, pid in enumerate(piece_ids)}

        def comp(S):
            c = 0
            for j in S:
                c ^= (M // 2) >> j
            return c

        desc = {}

        def start_piece(k, S):
            region = pl.multiple_of(own16 ^ comp(S), 8)
            d = pltpu.make_async_remote_copy(
                src_ref=out_ref.at[pl.ds(region, rows)],
                dst_ref=out_ref.at[pl.ds(region, rows)],
                send_sem=ag_send.at[sem_idx[(k, S)]],
                recv_sem=ag_recv.at[sem_idx[(k, S)]],
                device_id=(me ^ (1 << k),),
                device_id_type=pl.DeviceIdType.MESH,
            )
            d.start()
            desc[(k, S)] = d

        for k in range(LOG2 - 1, -1, -1):
            start_piece(k, ())

        for k, S in sorted(piece_ids, key=lambda p: (len(p[1]), -p[0])):
            desc[(k, S)].wait_recv()
            T = tuple(sorted(S + (k,)))
            for kp in range(k - 1, -1, -1):
                start_piece(kp, T)

        for d in desc.values():
            d.wait_send()

    return pl.pallas_call(
        body,
        out_shape=jax.ShapeDtypeStruct((M, D), jnp.float32),
        in_specs=[
            pl.BlockSpec(memory_space=pltpu.VMEM),
            pl.BlockSpec(memory_space=pltpu.VMEM),
            pl.BlockSpec(memory_space=pltpu.VMEM),
        ],
        out_specs=pl.BlockSpec(memory_space=pltpu.VMEM),
        scratch_shapes=[
            pltpu.VMEM((256, D), jnp.float32),
            pltpu.VMEM((128, D), jnp.float32),
            pltpu.VMEM((64, D), jnp.float32),
            pltpu.VMEM((32, D), jnp.float32),
            pltpu.VMEM((16, D), jnp.float32),
            pltpu.SemaphoreType.DMA((2 * LOG2 - 1,)),
            pltpu.SemaphoreType.DMA((2 * LOG2 - 1,)),
            pltpu.SemaphoreType.DMA((N_AG_SEMS,)),
            pltpu.SemaphoreType.DMA((N_AG_SEMS,)),
        ],
        compiler_params=pltpu.CompilerParams(collective_id=0),
    )(partial2d, resid, gamma2d)


# device time: 40779 ns/iter; 1.4114x vs baseline; 1.0009x over previous
from itertools import combinations

import jax
import jax.numpy as jnp
from jax import lax
from jax.experimental import pallas as pl
from jax.experimental.pallas import tpu as pltpu

N_DEV = 32
LOG2 = 5
M = 512
D = 512

N_AG_SEMS = 31


def kernel(partial, resid, gamma):
    partial2d = partial.reshape(M, D)
    gamma2d = gamma.reshape(1, D)

    def body(
        x_ref,
        resid_ref,
        gamma_ref,
        out_ref,
        rs_b0,
        rs_b1,
        rs_b2,
        rs_b3,
        rs_b4,
        resid16,
        rs_send,
        rs_recv,
        ag_send,
        ag_recv,
        resid_sem,
    ):
        me = lax.axis_index("i")
        rs_bufs = [rs_b0, rs_b1, rs_b2, rs_b3, rs_b4]
        rows = M // N_DEV

        own16 = jnp.int32(0)
        for k in range(LOG2):
            own16 = own16 + ((me >> k) & 1) * ((M // 2) >> k)
        own16 = pl.multiple_of(own16, 8)
        resid_cp = pltpu.make_async_copy(
            resid_ref.at[pl.ds(own16, rows)], resid16, resid_sem
        )
        resid_cp.start()

        barrier_sem = pltpu.get_barrier_semaphore()
        for k in range(LOG2):
            pl.semaphore_signal(
                barrier_sem,
                inc=1,
                device_id=(me ^ (1 << k),),
                device_id_type=pl.DeviceIdType.MESH,
            )
        pl.semaphore_wait(barrier_sem, LOG2)

        def issue_rs(k, seg_start):
            h = (M >> k) // 2
            partner = me ^ (1 << k)
            bit = (me >> k) & 1
            send_start = pl.multiple_of(
                jnp.where(bit == 0, seg_start + h, seg_start), 8
            )
            keep_start = pl.multiple_of(
                jnp.where(bit == 0, seg_start, seg_start + h), 8
            )
            src = x_ref if k == 0 else out_ref
            if k == LOG2 - 1:
                rdma = pltpu.make_async_remote_copy(
                    src_ref=src.at[pl.ds(send_start, h)],
                    dst_ref=rs_bufs[k],
                    send_sem=rs_send.at[2 * k],
                    recv_sem=rs_recv.at[2 * k],
                    device_id=(partner,),
                    device_id_type=pl.DeviceIdType.MESH,
                )
                rdma.start()
                return [(rdma, jnp.int32(0), h)], keep_start
            hb = h // 2
            bitn = (me >> (k + 1)) & 1
            off1 = jnp.where(bitn == 0, hb, 0)
            off2 = hb - off1
            subs = []
            for c, off in ((0, off1), (1, off2)):
                off = pl.multiple_of(off, 8)
                rdma = pltpu.make_async_remote_copy(
                    src_ref=src.at[pl.ds(send_start + off, hb)],
                    dst_ref=rs_bufs[k].at[pl.ds(c * hb, hb)],
                    send_sem=rs_send.at[2 * k + c],
                    recv_sem=rs_recv.at[2 * k + c],
                    device_id=(partner,),
                    device_id_type=pl.DeviceIdType.MESH,
                )
                rdma.start()
                subs.append((rdma, off, hb))
            return subs, keep_start

        rs_rdmas = []
        chunks, keep = issue_rs(0, jnp.int32(0))
        for k in range(LOG2):
            acc = x_ref if k == 0 else out_ref
            rdma1, o1, hb1 = chunks[0]
            rdma1.wait_recv()
            a1 = pl.multiple_of(keep + o1, 8)
            out_ref[pl.ds(a1, hb1)] = (
                acc[pl.ds(a1, hb1)] + rs_bufs[k][0:hb1, :]
            )
            rs_rdmas.append(rdma1)
            if k + 1 < LOG2:
                next_chunks, next_keep = issue_rs(k + 1, keep)
            if len(chunks) > 1:
                rdma2, o2, hb2 = chunks[1]
                rdma2.wait_recv()
                a2 = pl.multiple_of(keep + o2, 8)
                out_ref[pl.ds(a2, hb2)] = (
                    acc[pl.ds(a2, hb2)] + rs_bufs[k][hb2 : 2 * hb2, :]
                )
                rs_rdmas.append(rdma2)
            if k + 1 < LOG2:
                chunks, keep = next_chunks, next_keep
        seg_start = keep

        for rdma in rs_rdmas:
            rdma.wait_send()

        resid_cp.wait()
        y = out_ref[pl.ds(own16, rows)] + resid16[...]
        ms = jnp.mean(y * y, axis=-1, keepdims=True)
        out_ref[pl.ds(own16, rows)] = y * lax.rsqrt(ms + 1e-6) * gamma_ref[...]

        piece_ids = []
        for k in range(LOG2):
            for r in range(LOG2 - k):
                for S in combinations(range(k + 1, LOG2), r):
                    piece_ids.append((k, S))
        sem_idx = {pid: n for n, pid in enumerate(piece_ids)}

        def comp(S):
            c = 0
            for j in S:
                c ^= (M // 2) >> j
            return c

        desc = {}

        def start_piece(k, S):
            region = pl.multiple_of(own16 ^ comp(S), 8)
            d = pltpu.make_async_remote_copy(
                src_ref=out_ref.at[pl.ds(region, rows)],
                dst_ref=out_ref.at[pl.ds(region, rows)],
                send_sem=ag_send.at[sem_idx[(k, S)]],
                recv_sem=ag_recv.at[sem_idx[(k, S)]],
                device_id=(me ^ (1 << k),),
                device_id_type=pl.DeviceIdType.MESH,
            )
            d.start()
            desc[(k, S)] = d

        for k in range(LOG2 - 1, -1, -1):
            start_piece(k, ())

        for k, S in sorted(piece_ids, key=lambda p: (len(p[1]), -p[0])):
            desc[(k, S)].wait_recv()
            T = tuple(sorted(S + (k,)))
            for kp in range(k - 1, -1, -1):
                start_piece(kp, T)

        for d in desc.values():
            d.wait_send()

    return pl.pallas_call(
        body,
        out_shape=jax.ShapeDtypeStruct((M, D), jnp.float32),
        in_specs=[
            pl.BlockSpec(memory_space=pltpu.VMEM),
            pl.BlockSpec(memory_space=pl.ANY),
            pl.BlockSpec(memory_space=pltpu.VMEM),
        ],
        out_specs=pl.BlockSpec(memory_space=pltpu.VMEM),
        scratch_shapes=[
            pltpu.VMEM((256, D), jnp.float32),
            pltpu.VMEM((128, D), jnp.float32),
            pltpu.VMEM((64, D), jnp.float32),
            pltpu.VMEM((32, D), jnp.float32),
            pltpu.VMEM((16, D), jnp.float32),
            pltpu.VMEM((M // N_DEV, D), jnp.float32),
            pltpu.SemaphoreType.DMA((2 * LOG2 - 1,)),
            pltpu.SemaphoreType.DMA((2 * LOG2 - 1,)),
            pltpu.SemaphoreType.DMA((N_AG_SEMS,)),
            pltpu.SemaphoreType.DMA((N_AG_SEMS,)),
            pltpu.SemaphoreType.DMA,
        ],
        compiler_params=pltpu.CompilerParams(collective_id=0),
    )(partial2d, resid, gamma2d)


# device time: 37834 ns/iter; 1.5213x vs baseline; 1.0778x over previous
from itertools import combinations

import jax
import jax.numpy as jnp
from jax import lax
from jax.experimental import pallas as pl
from jax.experimental.pallas import tpu as pltpu

N_DEV = 32
LOG2 = 5
M = 512
D = 512

N_AG_SEMS = 31

PI = (0, 3, 1, 2, 4)


def kernel(partial, resid, gamma):
    partial2d = partial.reshape(M, D)
    gamma2d = gamma.reshape(1, D)

    def body(
        x_ref,
        resid_ref,
        gamma_ref,
        out_ref,
        rs_b0,
        rs_b1,
        rs_b2,
        rs_b3,
        rs_b4,
        resid16,
        rs_send,
        rs_recv,
        ag_send,
        ag_recv,
        resid_sem,
    ):
        me = lax.axis_index("i")
        rs_bufs = [rs_b0, rs_b1, rs_b2, rs_b3, rs_b4]
        rows = M // N_DEV

        own16 = jnp.int32(0)
        for k in range(LOG2):
            own16 = own16 + ((me >> PI[k]) & 1) * ((M // 2) >> k)
        own16 = pl.multiple_of(own16, 8)
        resid_cp = pltpu.make_async_copy(
            resid_ref.at[pl.ds(own16, rows)], resid16, resid_sem
        )
        resid_cp.start()

        barrier_sem = pltpu.get_barrier_semaphore()
        for k in range(LOG2):
            pl.semaphore_signal(
                barrier_sem,
                inc=1,
                device_id=(me ^ (1 << PI[k]),),
                device_id_type=pl.DeviceIdType.MESH,
            )
        pl.semaphore_wait(barrier_sem, LOG2)

        def issue_rs(k, seg_start):
            h = (M >> k) // 2
            partner = me ^ (1 << PI[k])
            bit = (me >> PI[k]) & 1
            send_start = pl.multiple_of(
                jnp.where(bit == 0, seg_start + h, seg_start), 8
            )
            keep_start = pl.multiple_of(
                jnp.where(bit == 0, seg_start, seg_start + h), 8
            )
            src = x_ref if k == 0 else out_ref
            if k == LOG2 - 1:
                rdma = pltpu.make_async_remote_copy(
                    src_ref=src.at[pl.ds(send_start, h)],
                    dst_ref=rs_bufs[k],
                    send_sem=rs_send.at[2 * k],
                    recv_sem=rs_recv.at[2 * k],
                    device_id=(partner,),
                    device_id_type=pl.DeviceIdType.MESH,
                )
                rdma.start()
                return [(rdma, jnp.int32(0), h)], keep_start
            hb = h // 2
            bitn = (me >> PI[k + 1]) & 1
            off1 = jnp.where(bitn == 0, hb, 0)
            off2 = hb - off1
            subs = []
            for c, off in ((0, off1), (1, off2)):
                off = pl.multiple_of(off, 8)
                rdma = pltpu.make_async_remote_copy(
                    src_ref=src.at[pl.ds(send_start + off, hb)],
                    dst_ref=rs_bufs[k].at[pl.ds(c * hb, hb)],
                    send_sem=rs_send.at[2 * k + c],
                    recv_sem=rs_recv.at[2 * k + c],
                    device_id=(partner,),
                    device_id_type=pl.DeviceIdType.MESH,
                )
                rdma.start()
                subs.append((rdma, off, hb))
            return subs, keep_start

        rs_rdmas = []
        chunks, keep = issue_rs(0, jnp.int32(0))
        for k in range(LOG2):
            acc = x_ref if k == 0 else out_ref
            rdma1, o1, hb1 = chunks[0]
            rdma1.wait_recv()
            a1 = pl.multiple_of(keep + o1, 8)
            out_ref[pl.ds(a1, hb1)] = (
                acc[pl.ds(a1, hb1)] + rs_bufs[k][0:hb1, :]
            )
            rs_rdmas.append(rdma1)
            if k + 1 < LOG2:
                next_chunks, next_keep = issue_rs(k + 1, keep)
            if len(chunks) > 1:
                rdma2, o2, hb2 = chunks[1]
                rdma2.wait_recv()
                a2 = pl.multiple_of(keep + o2, 8)
                out_ref[pl.ds(a2, hb2)] = (
                    acc[pl.ds(a2, hb2)] + rs_bufs[k][hb2 : 2 * hb2, :]
                )
                rs_rdmas.append(rdma2)
            if k + 1 < LOG2:
                chunks, keep = next_chunks, next_keep
        seg_start = keep

        for rdma in rs_rdmas:
            rdma.wait_send()

        resid_cp.wait()
        y = out_ref[pl.ds(own16, rows)] + resid16[...]
        ms = jnp.mean(y * y, axis=-1, keepdims=True)
        out_ref[pl.ds(own16, rows)] = y * lax.rsqrt(ms + 1e-6) * gamma_ref[...]

        piece_ids = []
        for k in range(LOG2):
            for r in range(LOG2 - k):
                for S in combinations(range(k + 1, LOG2), r):
                    piece_ids.append((k, S))
        sem_idx = {pid: n for n, pid in enumerate(piece_ids)}

        def comp(S):
            c = 0
            for j in S:
                c ^= (M // 2) >> j
            return c

        desc = {}

        def start_piece(k, S):
            region = pl.multiple_of(own16 ^ comp(S), 8)
            d = pltpu.make_async_remote_copy(
                src_ref=out_ref.at[pl.ds(region, rows)],
                dst_ref=out_ref.at[pl.ds(region, rows)],
                send_sem=ag_send.at[sem_idx[(k, S)]],
                recv_sem=ag_recv.at[sem_idx[(k, S)]],
                device_id=(me ^ (1 << PI[k]),),
                device_id_type=pl.DeviceIdType.MESH,
            )
            d.start()
            desc[(k, S)] = d

        for k in range(LOG2 - 1, -1, -1):
            start_piece(k, ())

        for k, S in sorted(piece_ids, key=lambda p: (len(p[1]), -p[0])):
            desc[(k, S)].wait_recv()
            T = tuple(sorted(S + (k,)))
            for kp in range(k - 1, -1, -1):
                start_piece(kp, T)

        for d in desc.values():
            d.wait_send()

    return pl.pallas_call(
        body,
        out_shape=jax.ShapeDtypeStruct((M, D), jnp.float32),
        in_specs=[
            pl.BlockSpec(memory_space=pltpu.VMEM),
            pl.BlockSpec(memory_space=pl.ANY),
            pl.BlockSpec(memory_space=pltpu.VMEM),
        ],
        out_specs=pl.BlockSpec(memory_space=pltpu.VMEM),
        scratch_shapes=[
            pltpu.VMEM((256, D), jnp.float32),
            pltpu.VMEM((128, D), jnp.float32),
            pltpu.VMEM((64, D), jnp.float32),
            pltpu.VMEM((32, D), jnp.float32),
            pltpu.VMEM((16, D), jnp.float32),
            pltpu.VMEM((M // N_DEV, D), jnp.float32),
            pltpu.SemaphoreType.DMA((2 * LOG2 - 1,)),
            pltpu.SemaphoreType.DMA((2 * LOG2 - 1,)),
            pltpu.SemaphoreType.DMA((N_AG_SEMS,)),
            pltpu.SemaphoreType.DMA((N_AG_SEMS,)),
            pltpu.SemaphoreType.DMA,
        ],
        compiler_params=pltpu.CompilerParams(collective_id=0),
    )(partial2d, resid, gamma2d)
